# baseline (speedup 1.0000x reference)
import jax
import jax.numpy as jnp
from jax.experimental import pallas as pl
from jax.experimental.pallas import tpu as pltpu

N = 10000
H = 128
BM = 200
BF = jnp.bfloat16


def _bdot(a, b):
    return jnp.dot(a.astype(BF), b.astype(BF),
                   preferred_element_type=jnp.float32)


def _mlp(x, m1_w, m1_b, m2_w, m2_b, m3_wr, m3_b):
    h1 = jnp.maximum(_bdot(x, m1_w) + m1_b, 0.0)
    h2 = jnp.maximum(_bdot(h1, m2_w) + m2_b, 0.0)
    prod = h2.astype(BF).astype(jnp.float32) * m3_wr
    return jnp.sum(prod, axis=1, keepdims=True) + m3_b


def _gcn_kernel(adj_ref, h0_ref, ws_ref, m1w_ref, m1b_ref,
                m2w_ref, m2b_ref, m3wr_ref, m3b_ref,
                s_ref, hbuf_ref):
    j = pl.program_id(0)
    i = pl.program_id(1)
    p = jax.lax.rem(j, 2)

    @pl.when(jnp.logical_and(j == 0, i == 0))
    def _():
        hbuf_ref[pl.ds(0, N), :] = h0_ref[...].astype(BF)

    src = p * N
    acc = jnp.dot(adj_ref[...].astype(BF), hbuf_ref[pl.ds(src, N), :],
                  preferred_element_type=jnp.float32)
    x = jnp.maximum(acc, 0.0)
    n = jnp.sqrt(jnp.sum(x * x, axis=1, keepdims=True))
    x = x / jnp.where(j < 6, jnp.maximum(n, 1e-12), 1.0)

    @pl.when(j < 6)
    def _():
        dst = (1 - p) * N
        hbuf_ref[pl.ds(dst + i * BM, BM), :] = _bdot(x, ws_ref[0]).astype(BF)

    s_ref[...] = _mlp(x, m1w_ref[...], m1b_ref[...], m2w_ref[...],
                      m2b_ref[...], m3wr_ref[...], m3b_ref[...])[None]


def _full(shape):
    return pl.BlockSpec(shape, lambda j, i: (0,) * len(shape))


def _branch(adj, w1, ws_stack, mlp_args, interpret=False):
    mlp_specs = [_full((H, H)), _full((1, H)), _full((H, H)),
                 _full((1, H)), _full((1, H)), _full((1, 1))]
    s_layers = pl.pallas_call(
        _gcn_kernel,
        grid=(7, N // BM),
        in_specs=[pl.BlockSpec((BM, N), lambda j, i: (i, 0)),
                  pl.BlockSpec((N, H), lambda j, i: (0, 0)),
                  pl.BlockSpec((1, H, H),
                               lambda j, i: (jnp.minimum(j, 5), 0, 0))]
                 + mlp_specs,
        out_specs=pl.BlockSpec((1, BM, 1), lambda j, i: (j, i, 0)),
        out_shape=jax.ShapeDtypeStruct((7, N, 1), jnp.float32),
        scratch_shapes=[pltpu.VMEM((2 * N, H), BF)],
        compiler_params=pltpu.CompilerParams(
            dimension_semantics=("arbitrary", "arbitrary")),
        interpret=interpret,
    )(adj, w1, ws_stack, *mlp_args)
    return jnp.sum(s_layers, axis=0)


def kernel(adj1, adj2, w1, w2, w3, w4, w5, w6, w7, m1_w, m1_b, m2_w, m2_b,
           m3_w, m3_b, interpret=False):
    ws_stack = jnp.stack([w2, w3, w4, w5, w6, w7])
    mlp_args = (m1_w, m1_b.reshape(1, H),
                m2_w, m2_b.reshape(1, H),
                m3_w.reshape(1, H),
                m3_b.reshape(1, 1))
    s1 = _branch(adj1, w1, ws_stack, mlp_args, interpret=interpret)
    s2 = _branch(adj2, w1, ws_stack, mlp_args, interpret=interpret)
    return s1 * s2


# speedup vs baseline: 1.0314x; 1.0314x over previous
import jax
import jax.numpy as jnp
from jax.experimental import pallas as pl
from jax.experimental.pallas import tpu as pltpu

N = 10000
H = 128
BM = 200
BF = jnp.bfloat16


def _bdot(a, b):
    return jnp.dot(a.astype(BF), b.astype(BF),
                   preferred_element_type=jnp.float32)


def _mlp(x, m1_w, m1_b, m2_w, m2_b, m3_wr, m3_b):
    h1 = jnp.maximum(_bdot(x, m1_w) + m1_b, 0.0)
    h2 = jnp.maximum(_bdot(h1, m2_w) + m2_b, 0.0)
    prod = h2.astype(BF).astype(jnp.float32) * m3_wr
    return jnp.sum(prod, axis=1, keepdims=True) + m3_b


def _gcn_kernel(adj_ref, h0_ref, ws_ref, m1w_ref, m1b_ref,
                m2w_ref, m2b_ref, m3wr_ref, m3b_ref,
                s_ref, hbuf_ref):
    j = pl.program_id(0)
    i = pl.program_id(1)
    p = jax.lax.rem(j, 2)

    @pl.when(jnp.logical_and(j == 0, i == 0))
    def _():
        hbuf_ref[pl.ds(0, N), :] = h0_ref[...].astype(BF)

    src = p * N
    acc = jnp.dot(adj_ref[...], hbuf_ref[pl.ds(src, N), :],
                  preferred_element_type=jnp.float32)
    x = jnp.maximum(acc, 0.0)
    n = jnp.sqrt(jnp.sum(x * x, axis=1, keepdims=True))
    x = x / jnp.where(j < 6, jnp.maximum(n, 1e-12), 1.0)

    @pl.when(j < 6)
    def _():
        dst = (1 - p) * N
        hbuf_ref[pl.ds(dst + i * BM, BM), :] = _bdot(x, ws_ref[0]).astype(BF)

    s_ref[...] = _mlp(x, m1w_ref[...], m1b_ref[...], m2w_ref[...],
                      m2b_ref[...], m3wr_ref[...], m3b_ref[...])[None]


def _full(shape):
    return pl.BlockSpec(shape, lambda j, i: (0,) * len(shape))


def _branch(adj, w1, ws_stack, mlp_args, interpret=False):
    mlp_specs = [_full((H, H)), _full((1, H)), _full((H, H)),
                 _full((1, H)), _full((1, H)), _full((1, 1))]
    s_layers = pl.pallas_call(
        _gcn_kernel,
        grid=(7, N // BM),
        in_specs=[pl.BlockSpec((BM, N), lambda j, i: (i, 0)),
                  pl.BlockSpec((N, H), lambda j, i: (0, 0)),
                  pl.BlockSpec((1, H, H),
                               lambda j, i: (jnp.minimum(j, 5), 0, 0))]
                 + mlp_specs,
        out_specs=pl.BlockSpec((1, BM, 1), lambda j, i: (j, i, 0)),
        out_shape=jax.ShapeDtypeStruct((7, N, 1), jnp.float32),
        scratch_shapes=[pltpu.VMEM((2 * N, H), BF)],
        compiler_params=pltpu.CompilerParams(
            dimension_semantics=("arbitrary", "arbitrary")),
        interpret=interpret,
    )(adj, w1, ws_stack, *mlp_args)
    return jnp.sum(s_layers, axis=0)


def kernel(adj1, adj2, w1, w2, w3, w4, w5, w6, w7, m1_w, m1_b, m2_w, m2_b,
           m3_w, m3_b, interpret=False):
    ws_stack = jnp.stack([w2, w3, w4, w5, w6, w7])
    mlp_args = (m1_w, m1_b.reshape(1, H),
                m2_w, m2_b.reshape(1, H),
                m3_w.reshape(1, H),
                m3_b.reshape(1, 1))
    s1 = _branch(adj1.astype(BF), w1, ws_stack, mlp_args, interpret=interpret)
    s2 = _branch(adj2.astype(BF), w1, ws_stack, mlp_args, interpret=interpret)
    return s1 * s2


# speedup vs baseline: 1.2506x; 1.2126x over previous
import jax
import jax.numpy as jnp
from jax.experimental import pallas as pl
from jax.experimental.pallas import tpu as pltpu

N = 10000
H = 128
BM = 400
BF = jnp.bfloat16


def _bdot(a, b):
    return jnp.dot(a.astype(BF), b.astype(BF),
                   preferred_element_type=jnp.float32)


def _mlp(x, m1_w, m1_b, m2_w, m2_b, m3_wr, m3_b):
    h1 = jnp.maximum(_bdot(x, m1_w) + m1_b, 0.0)
    h2 = jnp.maximum(_bdot(h1, m2_w) + m2_b, 0.0)
    prod = h2.astype(BF).astype(jnp.float32) * m3_wr
    return jnp.sum(prod, axis=1, keepdims=True) + m3_b


def _gcn_kernel(adj_ref, h0_ref, ws_ref, m1w_ref, m1b_ref,
                m2w_ref, m2b_ref, m3wr_ref, m3b_ref,
                s_ref, hbuf_ref):
    j = pl.program_id(0)
    i = pl.program_id(1)
    p = jax.lax.rem(j, 2)

    @pl.when(jnp.logical_and(j == 0, i == 0))
    def _():
        hbuf_ref[pl.ds(0, N), :] = h0_ref[...].astype(BF)

    src = p * N
    acc = jnp.dot(adj_ref[...], hbuf_ref[pl.ds(src, N), :],
                  preferred_element_type=jnp.float32)
    x = jnp.maximum(acc, 0.0)
    n = jnp.sqrt(jnp.sum(x * x, axis=1, keepdims=True))
    x = x / jnp.where(j < 6, jnp.maximum(n, 1e-12), 1.0)

    @pl.when(j < 6)
    def _():
        dst = (1 - p) * N
        hbuf_ref[pl.ds(dst + i * BM, BM), :] = _bdot(x, ws_ref[0]).astype(BF)

    s_ref[...] = _mlp(x, m1w_ref[...], m1b_ref[...], m2w_ref[...],
                      m2b_ref[...], m3wr_ref[...], m3b_ref[...])[None]


def _full(shape):
    return pl.BlockSpec(shape, lambda j, i: (0,) * len(shape))


def _branch(adj, w1, ws_stack, mlp_args, interpret=False):
    mlp_specs = [_full((H, H)), _full((1, H)), _full((H, H)),
                 _full((1, H)), _full((1, H)), _full((1, 1))]
    s_layers = pl.pallas_call(
        _gcn_kernel,
        grid=(7, N // BM),
        in_specs=[pl.BlockSpec((BM, N), lambda j, i: (i, 0)),
                  pl.BlockSpec((N, H), lambda j, i: (0, 0)),
                  pl.BlockSpec((1, H, H),
                               lambda j, i: (jnp.minimum(j, 5), 0, 0))]
                 + mlp_specs,
        out_specs=pl.BlockSpec((1, BM, 1), lambda j, i: (j, i, 0)),
        out_shape=jax.ShapeDtypeStruct((7, N, 1), jnp.float32),
        scratch_shapes=[pltpu.VMEM((2 * N, H), BF)],
        compiler_params=pltpu.CompilerParams(
            dimension_semantics=("arbitrary", "arbitrary")),
        interpret=interpret,
    )(adj, w1, ws_stack, *mlp_args)
    return jnp.sum(s_layers, axis=0)


def kernel(adj1, adj2, w1, w2, w3, w4, w5, w6, w7, m1_w, m1_b, m2_w, m2_b,
           m3_w, m3_b, interpret=False):
    ws_stack = jnp.stack([w2, w3, w4, w5, w6, w7])
    mlp_args = (m1_w, m1_b.reshape(1, H),
                m2_w, m2_b.reshape(1, H),
                m3_w.reshape(1, H),
                m3_b.reshape(1, 1))
    s1 = _branch(adj1.astype(BF), w1, ws_stack, mlp_args, interpret=interpret)
    s2 = _branch(adj2.astype(BF), w1, ws_stack, mlp_args, interpret=interpret)
    return s1 * s2


# speedup vs baseline: 1.4106x; 1.1280x over previous
import jax
import jax.numpy as jnp
from jax.experimental import pallas as pl
from jax.experimental.pallas import tpu as pltpu

N = 10000
H = 128
BM = 1000
BF = jnp.bfloat16


def _bdot(a, b):
    return jnp.dot(a.astype(BF), b.astype(BF),
                   preferred_element_type=jnp.float32)


def _mlp(x, m1_w, m1_b, m2_w, m2_b, m3_wr, m3_b):
    h1 = jnp.maximum(_bdot(x, m1_w) + m1_b, 0.0)
    h2 = jnp.maximum(_bdot(h1, m2_w) + m2_b, 0.0)
    prod = h2.astype(BF).astype(jnp.float32) * m3_wr
    return jnp.sum(prod, axis=1, keepdims=True) + m3_b


def _gcn_kernel(adj_ref, h0_ref, ws_ref, m1w_ref, m1b_ref,
                m2w_ref, m2b_ref, m3wr_ref, m3b_ref,
                s_ref, hbuf_ref):
    j = pl.program_id(0)
    i = pl.program_id(1)
    p = jax.lax.rem(j, 2)

    @pl.when(jnp.logical_and(j == 0, i == 0))
    def _():
        hbuf_ref[pl.ds(0, N), :] = h0_ref[...].astype(BF)

    src = p * N
    acc = jnp.dot(adj_ref[...], hbuf_ref[pl.ds(src, N), :],
                  preferred_element_type=jnp.float32)
    x = jnp.maximum(acc, 0.0)
    n = jnp.sqrt(jnp.sum(x * x, axis=1, keepdims=True))
    x = x / jnp.where(j < 6, jnp.maximum(n, 1e-12), 1.0)

    @pl.when(j < 6)
    def _():
        dst = (1 - p) * N
        hbuf_ref[pl.ds(dst + i * BM, BM), :] = _bdot(x, ws_ref[0]).astype(BF)

    s_ref[...] = _mlp(x, m1w_ref[...], m1b_ref[...], m2w_ref[...],
                      m2b_ref[...], m3wr_ref[...], m3b_ref[...])[None]


def _full(shape):
    return pl.BlockSpec(shape, lambda j, i: (0,) * len(shape))


def _branch(adj, w1, ws_stack, mlp_args, interpret=False):
    mlp_specs = [_full((H, H)), _full((1, H)), _full((H, H)),
                 _full((1, H)), _full((1, H)), _full((1, 1))]
    s_layers = pl.pallas_call(
        _gcn_kernel,
        grid=(7, N // BM),
        in_specs=[pl.BlockSpec((BM, N), lambda j, i: (i, 0)),
                  pl.BlockSpec((N, H), lambda j, i: (0, 0)),
                  pl.BlockSpec((1, H, H),
                               lambda j, i: (jnp.minimum(j, 5), 0, 0))]
                 + mlp_specs,
        out_specs=pl.BlockSpec((1, BM, 1), lambda j, i: (j, i, 0)),
        out_shape=jax.ShapeDtypeStruct((7, N, 1), jnp.float32),
        scratch_shapes=[pltpu.VMEM((2 * N, H), BF)],
        compiler_params=pltpu.CompilerParams(
            dimension_semantics=("arbitrary", "arbitrary")),
        interpret=interpret,
    )(adj, w1, ws_stack, *mlp_args)
    return jnp.sum(s_layers, axis=0)


def kernel(adj1, adj2, w1, w2, w3, w4, w5, w6, w7, m1_w, m1_b, m2_w, m2_b,
           m3_w, m3_b, interpret=False):
    ws_stack = jnp.stack([w2, w3, w4, w5, w6, w7])
    mlp_args = (m1_w, m1_b.reshape(1, H),
                m2_w, m2_b.reshape(1, H),
                m3_w.reshape(1, H),
                m3_b.reshape(1, 1))
    s1 = _branch(adj1.astype(BF), w1, ws_stack, mlp_args, interpret=interpret)
    s2 = _branch(adj2.astype(BF), w1, ws_stack, mlp_args, interpret=interpret)
    return s1 * s2


# speedup vs baseline: 1.5211x; 1.0783x over previous
import functools

import jax
import jax.numpy as jnp
from jax.experimental import pallas as pl
from jax.experimental.pallas import tpu as pltpu

N = 10000
H = 128
BM = 1000
BF = jnp.bfloat16


def _bdot(a, b):
    return jnp.dot(a.astype(BF), b.astype(BF),
                   preferred_element_type=jnp.float32)


def _mlp(x, m1_w, m1_b, m2_w, m2_b, m3_wr, m3_b):
    h1 = jnp.maximum(_bdot(x, m1_w) + m1_b, 0.0)
    h2 = jnp.maximum(_bdot(h1, m2_w) + m2_b, 0.0)
    prod = h2.astype(BF).astype(jnp.float32) * m3_wr
    return jnp.sum(prod, axis=1, keepdims=True) + m3_b


BM0 = 200


def _layer0_kernel(adj_ref, h0_ref, w2_ref, m1w_ref, m1b_ref,
                   m2w_ref, m2b_ref, m3wr_ref, m3b_ref,
                   s_ref, h1_ref, adjb_ref):
    adjb = adj_ref[...].astype(BF)
    adjb_ref[...] = adjb
    acc = jnp.dot(adjb, h0_ref[...].astype(BF),
                  preferred_element_type=jnp.float32)
    x = jnp.maximum(acc, 0.0)
    n = jnp.sqrt(jnp.sum(x * x, axis=1, keepdims=True))
    x = x / jnp.maximum(n, 1e-12)
    h1_ref[...] = _bdot(x, w2_ref[...]).astype(BF)
    s_ref[...] = _mlp(x, m1w_ref[...], m1b_ref[...], m2w_ref[...],
                      m2b_ref[...], m3wr_ref[...], m3b_ref[...])


def _gcn_kernel(adj_ref, h0_ref, ws_ref, m1w_ref, m1b_ref,
                m2w_ref, m2b_ref, m3wr_ref, m3b_ref,
                s_ref, hbuf_ref, *, nlayers):
    j = pl.program_id(0)
    i = pl.program_id(1)
    p = jax.lax.rem(j, 2)

    @pl.when(jnp.logical_and(j == 0, i == 0))
    def _():
        hbuf_ref[pl.ds(0, N), :] = h0_ref[...].astype(BF)

    src = p * N
    acc = jnp.dot(adj_ref[...], hbuf_ref[pl.ds(src, N), :],
                  preferred_element_type=jnp.float32)
    x = jnp.maximum(acc, 0.0)
    n = jnp.sqrt(jnp.sum(x * x, axis=1, keepdims=True))
    x = x / jnp.where(j < nlayers - 1, jnp.maximum(n, 1e-12), 1.0)

    @pl.when(j < nlayers - 1)
    def _():
        dst = (1 - p) * N
        hbuf_ref[pl.ds(dst + i * BM, BM), :] = _bdot(x, ws_ref[0]).astype(BF)

    s_ref[...] = _mlp(x, m1w_ref[...], m1b_ref[...], m2w_ref[...],
                      m2b_ref[...], m3wr_ref[...], m3b_ref[...])[None]


def _full(shape):
    return pl.BlockSpec(shape, lambda j, i: (0,) * len(shape))


def _full0(shape):
    return pl.BlockSpec(shape, lambda i: (0,) * len(shape))


def _branch(adj, w1, w2, ws_stack, mlp_args, interpret=False):
    mlp_specs0 = [_full0((H, H)), _full0((1, H)), _full0((H, H)),
                  _full0((1, H)), _full0((1, H)), _full0((1, 1))]
    s0, h1, adjb = pl.pallas_call(
        _layer0_kernel,
        grid=(N // BM0,),
        in_specs=[pl.BlockSpec((BM0, N), lambda i: (i, 0)),
                  pl.BlockSpec((N, H), lambda i: (0, 0)),
                  _full0((H, H))] + mlp_specs0,
        out_specs=[pl.BlockSpec((BM0, 1), lambda i: (i, 0)),
                   pl.BlockSpec((BM0, H), lambda i: (i, 0)),
                   pl.BlockSpec((BM0, N), lambda i: (i, 0))],
        out_shape=[jax.ShapeDtypeStruct((N, 1), jnp.float32),
                   jax.ShapeDtypeStruct((N, H), BF),
                   jax.ShapeDtypeStruct((N, N), BF)],
        compiler_params=pltpu.CompilerParams(
            dimension_semantics=("arbitrary",)),
        interpret=interpret,
    )(adj, w1, w2, *mlp_args)

    mlp_specs = [_full((H, H)), _full((1, H)), _full((H, H)),
                 _full((1, H)), _full((1, H)), _full((1, 1))]
    s_layers = pl.pallas_call(
        functools.partial(_gcn_kernel, nlayers=6),
        grid=(6, N // BM),
        in_specs=[pl.BlockSpec((BM, N), lambda j, i: (i, 0)),
                  pl.BlockSpec((N, H), lambda j, i: (0, 0)),
                  pl.BlockSpec((1, H, H),
                               lambda j, i: (jnp.minimum(j, 4), 0, 0))]
                 + mlp_specs,
        out_specs=pl.BlockSpec((1, BM, 1), lambda j, i: (j, i, 0)),
        out_shape=jax.ShapeDtypeStruct((6, N, 1), jnp.float32),
        scratch_shapes=[pltpu.VMEM((2 * N, H), BF)],
        compiler_params=pltpu.CompilerParams(
            dimension_semantics=("arbitrary", "arbitrary")),
        interpret=interpret,
    )(adjb, h1, ws_stack, *mlp_args)
    return s0 + jnp.sum(s_layers, axis=0)


def kernel(adj1, adj2, w1, w2, w3, w4, w5, w6, w7, m1_w, m1_b, m2_w, m2_b,
           m3_w, m3_b, interpret=False):
    ws_stack = jnp.stack([w3, w4, w5, w6, w7])
    mlp_args = (m1_w, m1_b.reshape(1, H),
                m2_w, m2_b.reshape(1, H),
                m3_w.reshape(1, H),
                m3_b.reshape(1, 1))
    s1 = _branch(adj1, w1, w2, ws_stack, mlp_args, interpret=interpret)
    s2 = _branch(adj2, w1, w2, ws_stack, mlp_args, interpret=interpret)
    return s1 * s2
